# Initial kernel scaffold; baseline (speedup 1.0000x reference)
import jax
import jax.numpy as jnp
from jax.experimental import pallas as pl


def kernel(x, atom_to_cycle, emb_weight):
    raise NotImplementedError("write your pallas kernel here")


# speedup vs baseline: 26.9698x; 26.9698x over previous
import functools

import jax
import jax.numpy as jnp
from jax import lax
from jax.experimental import pallas as pl
from jax.experimental.pallas import tpu as pltpu
from jax.experimental.pallas import tpu_sc as plsc

N_ATOMS = 10000
N_MAP = 320000
NUM_CYCLES = 10000
HIDDEN_DIM = 128
VOCAB = 22
KPAD = 32
NC, NS, L = 2, 16, 16
NW = NC * NS
CHUNK = N_MAP // NW
HBINS = NUM_CYCLES * KPAD
ZCHUNK = HBINS // NS


def _sc_hist_body(x_hbm, src_hbm, dst_hbm, z_hbm, ones_hbm, hist_hbm,
                  x_v, src_v, dst_v, key_v, ones_v, stage_v, hist_sp):
    c = lax.axis_index("c")
    s = lax.axis_index("s")
    w = c * NS + s
    base = w * CHUNK

    pltpu.sync_copy(z_hbm, stage_v)
    pltpu.sync_copy(stage_v, hist_sp.at[pl.ds(s * ZCHUNK, ZCHUNK)])
    pltpu.sync_copy(x_hbm, x_v)
    pltpu.sync_copy(src_hbm.at[pl.ds(base, CHUNK)], src_v)
    pltpu.sync_copy(dst_hbm.at[pl.ds(base, CHUNK)], dst_v)
    pltpu.sync_copy(ones_hbm, ones_v)

    def body(i, carry):
        sv = src_v[pl.ds(i * L, L)]
        tv = plsc.load_gather(x_v, [sv])
        dv = dst_v[pl.ds(i * L, L)]
        key_v[pl.ds(i * L, L)] = dv * KPAD + tv
        return carry

    lax.fori_loop(0, CHUNK // L, body, 0)

    plsc.subcore_barrier()
    pltpu.sync_copy(ones_v, hist_sp.at[key_v], add=True)
    plsc.subcore_barrier()

    pltpu.sync_copy(hist_sp.at[pl.ds(s * ZCHUNK, ZCHUNK)], stage_v)
    pltpu.sync_copy(stage_v,
                    hist_hbm.at[pl.ds(c * HBINS + s * ZCHUNK, ZCHUNK)])


_SC_HIST_CACHE = []


def _sc_hist(*args):
    if not _SC_HIST_CACHE:
        _SC_HIST_CACHE.append(functools.partial(
            pl.kernel,
            out_type=jax.ShapeDtypeStruct((NC * HBINS,), jnp.float32),
            mesh=plsc.VectorSubcoreMesh(core_axis_name="c",
                                        subcore_axis_name="s",
                                        num_cores=NC, num_subcores=NS),
            compiler_params=pltpu.CompilerParams(needs_layout_passes=False),
            scratch_types=[
                pltpu.VMEM((N_ATOMS,), jnp.int32),
                pltpu.VMEM((CHUNK,), jnp.int32),
                pltpu.VMEM((CHUNK,), jnp.int32),
                pltpu.VMEM((CHUNK,), jnp.int32),
                pltpu.VMEM((CHUNK,), jnp.float32),
                pltpu.VMEM((ZCHUNK,), jnp.float32),
                pltpu.VMEM_SHARED((HBINS,), jnp.float32),
            ],
        )(_sc_hist_body))
    return _SC_HIST_CACHE[0](*args)


def _mm_body(h_ref, e_ref, o_ref):
    o_ref[...] = jnp.dot(h_ref[0] + h_ref[1], e_ref[...],
                         preferred_element_type=jnp.float32)


def _tc_matmul(h3, emb_pad):
    rows = 1000
    return pl.pallas_call(
        _mm_body,
        grid=(NUM_CYCLES // rows,),
        in_specs=[
            pl.BlockSpec((NC, rows, KPAD), lambda i: (0, i, 0)),
            pl.BlockSpec((KPAD, HIDDEN_DIM), lambda i: (0, 0)),
        ],
        out_specs=pl.BlockSpec((rows, HIDDEN_DIM), lambda i: (i, 0)),
        out_shape=jax.ShapeDtypeStruct((NUM_CYCLES, HIDDEN_DIM), jnp.float32),
    )(h3, emb_pad)


def kernel(x, atom_to_cycle, emb_weight):
    x = x.astype(jnp.int32)
    a2c = atom_to_cycle.astype(jnp.int32)
    zeros = jnp.zeros((ZCHUNK,), jnp.float32)
    ones = jnp.ones((CHUNK,), jnp.float32)
    hist = _sc_hist(x, a2c[0], a2c[1], zeros, ones)
    h3 = hist.reshape(NC, NUM_CYCLES, KPAD)
    emb_pad = jnp.concatenate(
        [emb_weight.astype(jnp.float32),
         jnp.zeros((KPAD - VOCAB, HIDDEN_DIM), jnp.float32)], axis=0)
    return _tc_matmul(h3, emb_pad)


# speedup vs baseline: 31.5121x; 1.1684x over previous
import functools

import jax
import jax.numpy as jnp
from jax import lax
from jax.experimental import pallas as pl
from jax.experimental.pallas import tpu as pltpu
from jax.experimental.pallas import tpu_sc as plsc

N_ATOMS = 10000
N_MAP = 320000
NUM_CYCLES = 10000
HIDDEN_DIM = 128
VOCAB = 22
KPAD = 32
NC, NS, L = 2, 16, 16
NW = NC * NS
CHUNK = N_MAP // NW
HBINS = NUM_CYCLES * KPAD
ZCHUNK = HBINS // NS
NCHUNK = 5
SUB = CHUNK // NCHUNK
SUBV = SUB // L


def _sc_hist_body(x_hbm, a2c_hbm, z_hbm, ones_hbm, hist_hbm,
                  x_v, src_v, dst_v, key_v, ones_v, stage_v, hist_sp):
    c = lax.axis_index("c")
    s = lax.axis_index("s")
    base = (c * NS + s) * CHUNK

    pltpu.sync_copy(z_hbm, stage_v)
    pltpu.sync_copy(stage_v, hist_sp.at[pl.ds(s * ZCHUNK, ZCHUNK)])
    pltpu.sync_copy(x_hbm, x_v)
    pltpu.sync_copy(a2c_hbm.at[pl.ds(base, CHUNK)], src_v)
    pltpu.sync_copy(a2c_hbm.at[pl.ds(N_MAP + base, CHUNK)], dst_v)
    pltpu.sync_copy(ones_hbm, ones_v)

    def body(i, carry):
        sv = src_v[pl.ds(i * L, L)]
        tv = plsc.load_gather(x_v, [sv])
        dv = dst_v[pl.ds(i * L, L)]
        key_v[pl.ds(i * L, L)] = dv * KPAD + tv
        return carry

    lax.fori_loop(0, CHUNK // L, body, 0)

    plsc.subcore_barrier()
    pltpu.sync_copy(ones_v, hist_sp.at[key_v], add=True)
    plsc.subcore_barrier()

    pltpu.sync_copy(hist_sp.at[pl.ds(s * ZCHUNK, ZCHUNK)], stage_v)
    pltpu.sync_copy(stage_v,
                    hist_hbm.at[pl.ds(c * HBINS + s * ZCHUNK, ZCHUNK)])


_SC_HIST_CACHE = []


def _sc_hist(*args):
    if not _SC_HIST_CACHE:
        _SC_HIST_CACHE.append(functools.partial(
            pl.kernel,
            out_type=jax.ShapeDtypeStruct((NC * HBINS,), jnp.float32),
            mesh=plsc.VectorSubcoreMesh(core_axis_name="c",
                                        subcore_axis_name="s",
                                        num_cores=NC, num_subcores=NS),
            compiler_params=pltpu.CompilerParams(needs_layout_passes=False),
            scratch_types=[
                pltpu.VMEM((N_ATOMS,), jnp.int32),
                pltpu.VMEM((CHUNK,), jnp.int32),
                pltpu.VMEM((CHUNK,), jnp.int32),
                pltpu.VMEM((CHUNK,), jnp.int32),
                pltpu.VMEM((CHUNK,), jnp.float32),
                pltpu.VMEM((ZCHUNK,), jnp.float32),
                pltpu.VMEM_SHARED((HBINS,), jnp.float32),
            ],
        )(_sc_hist_body))
    return _SC_HIST_CACHE[0](*args)


def _mm_body(h_ref, e_ref, o_ref):
    o_ref[...] = jnp.dot(h_ref[0] + h_ref[1], e_ref[...],
                         preferred_element_type=jnp.float32)


def _tc_matmul(h3, emb_pad):
    rows = 1000
    return pl.pallas_call(
        _mm_body,
        grid=(NUM_CYCLES // rows,),
        in_specs=[
            pl.BlockSpec((NC, rows, KPAD), lambda i: (0, i, 0)),
            pl.BlockSpec((KPAD, HIDDEN_DIM), lambda i: (0, 0)),
        ],
        out_specs=pl.BlockSpec((rows, HIDDEN_DIM), lambda i: (i, 0)),
        out_shape=jax.ShapeDtypeStruct((NUM_CYCLES, HIDDEN_DIM), jnp.float32),
    )(h3, emb_pad)


def kernel(x, atom_to_cycle, emb_weight):
    x = x.astype(jnp.int32)
    a2c_flat = atom_to_cycle.astype(jnp.int32).reshape(2 * N_MAP)
    zeros = jnp.zeros((ZCHUNK,), jnp.float32)
    ones = jnp.ones((CHUNK,), jnp.float32)
    hist = _sc_hist(x, a2c_flat, zeros, ones)
    h3 = hist.reshape(NC, NUM_CYCLES, KPAD)
    emb_pad = jnp.concatenate(
        [emb_weight.astype(jnp.float32),
         jnp.zeros((KPAD - VOCAB, HIDDEN_DIM), jnp.float32)], axis=0)
    return _tc_matmul(h3, emb_pad)


# speedup vs baseline: 33.2122x; 1.0540x over previous
import functools

import jax
import jax.numpy as jnp
from jax import lax
from jax.experimental import pallas as pl
from jax.experimental.pallas import tpu as pltpu
from jax.experimental.pallas import tpu_sc as plsc

N_ATOMS = 10000
N_MAP = 320000
NUM_CYCLES = 10000
HIDDEN_DIM = 128
VOCAB = 22
KPAD = 32
NC, NS, L = 2, 16, 16
NW = NC * NS
CHUNK = N_MAP // NW
HBINS = NUM_CYCLES * KPAD
ZCHUNK = HBINS // NS
NCHUNK = 5
SUB = CHUNK // NCHUNK
SUBV = SUB // L


def _sc_hist_body(x_hbm, a2c_hbm, z_hbm, ones_hbm, hist_hbm,
                  x_v, src_v, dst_v, key_v, ones_v, stage_v, hist_sp,
                  sem_in):
    c = lax.axis_index("c")
    s = lax.axis_index("s")
    base = (c * NS + s) * CHUNK

    cp_z = pltpu.async_copy(z_hbm, stage_v, sem_in)
    cp_x = pltpu.async_copy(x_hbm, x_v, sem_in)
    cp_s = pltpu.async_copy(a2c_hbm.at[pl.ds(base, CHUNK)], src_v, sem_in)
    cp_d = pltpu.async_copy(a2c_hbm.at[pl.ds(N_MAP + base, CHUNK)], dst_v,
                            sem_in)
    cp_o = pltpu.async_copy(ones_hbm, ones_v, sem_in)
    cp_z.wait()
    pltpu.sync_copy(stage_v, hist_sp.at[pl.ds(s * ZCHUNK, ZCHUNK)])
    cp_x.wait()
    cp_s.wait()
    cp_d.wait()
    cp_o.wait()

    def body(i, carry):
        sv = src_v[pl.ds(i * L, L)]
        tv = plsc.load_gather(x_v, [sv])
        dv = dst_v[pl.ds(i * L, L)]
        key_v[pl.ds(i * L, L)] = dv * KPAD + tv
        return carry

    lax.fori_loop(0, CHUNK // L, body, 0)

    plsc.subcore_barrier()
    pltpu.sync_copy(ones_v, hist_sp.at[key_v], add=True)
    plsc.subcore_barrier()

    pltpu.sync_copy(hist_sp.at[pl.ds(s * ZCHUNK, ZCHUNK)], stage_v)
    pltpu.sync_copy(stage_v,
                    hist_hbm.at[pl.ds(c * HBINS + s * ZCHUNK, ZCHUNK)])


_SC_HIST_CACHE = []


def _sc_hist(*args):
    if not _SC_HIST_CACHE:
        _SC_HIST_CACHE.append(functools.partial(
            pl.kernel,
            out_type=jax.ShapeDtypeStruct((NC * HBINS,), jnp.float32),
            mesh=plsc.VectorSubcoreMesh(core_axis_name="c",
                                        subcore_axis_name="s",
                                        num_cores=NC, num_subcores=NS),
            compiler_params=pltpu.CompilerParams(needs_layout_passes=False),
            scratch_types=[
                pltpu.VMEM((N_ATOMS,), jnp.int32),
                pltpu.VMEM((CHUNK,), jnp.int32),
                pltpu.VMEM((CHUNK,), jnp.int32),
                pltpu.VMEM((CHUNK,), jnp.int32),
                pltpu.VMEM((CHUNK,), jnp.float32),
                pltpu.VMEM((ZCHUNK,), jnp.float32),
                pltpu.VMEM_SHARED((HBINS,), jnp.float32),
                pltpu.SemaphoreType.DMA,
            ],
        )(_sc_hist_body))
    return _SC_HIST_CACHE[0](*args)


def _mm_body(h_ref, e_ref, o_ref):
    o_ref[...] = jnp.dot(h_ref[0] + h_ref[1], e_ref[...],
                         preferred_element_type=jnp.float32)


def _tc_matmul(h3, emb_pad):
    rows = 1000
    return pl.pallas_call(
        _mm_body,
        grid=(NUM_CYCLES // rows,),
        in_specs=[
            pl.BlockSpec((NC, rows, KPAD), lambda i: (0, i, 0)),
            pl.BlockSpec((KPAD, HIDDEN_DIM), lambda i: (0, 0)),
        ],
        out_specs=pl.BlockSpec((rows, HIDDEN_DIM), lambda i: (i, 0)),
        out_shape=jax.ShapeDtypeStruct((NUM_CYCLES, HIDDEN_DIM), jnp.float32),
    )(h3, emb_pad)


def kernel(x, atom_to_cycle, emb_weight):
    x = x.astype(jnp.int32)
    a2c_flat = atom_to_cycle.astype(jnp.int32).reshape(2 * N_MAP)
    zeros = jnp.zeros((ZCHUNK,), jnp.float32)
    ones = jnp.ones((CHUNK,), jnp.float32)
    hist = _sc_hist(x, a2c_flat, zeros, ones)
    h3 = hist.reshape(NC, NUM_CYCLES, KPAD)
    emb_pad = jnp.concatenate(
        [emb_weight.astype(jnp.float32),
         jnp.zeros((KPAD - VOCAB, HIDDEN_DIM), jnp.float32)], axis=0)
    return _tc_matmul(h3, emb_pad)


# speedup vs baseline: 33.4796x; 1.0081x over previous
import functools

import jax
import jax.numpy as jnp
from jax import lax
from jax.experimental import pallas as pl
from jax.experimental.pallas import tpu as pltpu
from jax.experimental.pallas import tpu_sc as plsc

N_ATOMS = 10000
N_MAP = 320000
NUM_CYCLES = 10000
HIDDEN_DIM = 128
VOCAB = 22
KPAD = 32
NC, NS, L = 2, 16, 16
NW = NC * NS
CHUNK = N_MAP // NW
HBINS = NUM_CYCLES * KPAD
ZCHUNK = HBINS // NS
NCHUNK = 5
SUB = CHUNK // NCHUNK
SUBV = SUB // L


def _sc_hist_body(x_hbm, a2c_hbm, z_hbm, ones_hbm, hist_hbm,
                  x_v, src_v, dst_v, key_v, ones_v, stage_v, hist_sp,
                  sem_in):
    c = lax.axis_index("c")
    s = lax.axis_index("s")
    base = (c * NS + s) * CHUNK

    cp_z = pltpu.async_copy(z_hbm, stage_v, sem_in)
    cp_x = pltpu.async_copy(x_hbm, x_v, sem_in)
    cp_s = pltpu.async_copy(a2c_hbm.at[pl.ds(base, CHUNK)], src_v, sem_in)
    cp_d = pltpu.async_copy(a2c_hbm.at[pl.ds(N_MAP + base, CHUNK)], dst_v,
                            sem_in)
    cp_o = pltpu.async_copy(ones_hbm, ones_v, sem_in)
    cp_z.wait()
    pltpu.sync_copy(stage_v, hist_sp.at[pl.ds(s * ZCHUNK, ZCHUNK)])
    cp_x.wait()
    cp_s.wait()
    cp_d.wait()
    cp_o.wait()

    def body(i, carry):
        for u in range(5):
            off = i * 5 * L + u * L
            sv = src_v[pl.ds(off, L)]
            tv = plsc.load_gather(x_v, [sv])
            dv = dst_v[pl.ds(off, L)]
            key_v[pl.ds(off, L)] = dv * KPAD + tv
        return carry

    lax.fori_loop(0, CHUNK // (5 * L), body, 0)

    plsc.subcore_barrier()
    pltpu.sync_copy(ones_v, hist_sp.at[key_v], add=True)
    plsc.subcore_barrier()

    pltpu.sync_copy(hist_sp.at[pl.ds(s * ZCHUNK, ZCHUNK)], stage_v)
    pltpu.sync_copy(stage_v,
                    hist_hbm.at[pl.ds(c * HBINS + s * ZCHUNK, ZCHUNK)])


_SC_HIST_CACHE = []


def _sc_hist(*args):
    if not _SC_HIST_CACHE:
        _SC_HIST_CACHE.append(functools.partial(
            pl.kernel,
            out_type=jax.ShapeDtypeStruct((NC * HBINS,), jnp.float32),
            mesh=plsc.VectorSubcoreMesh(core_axis_name="c",
                                        subcore_axis_name="s",
                                        num_cores=NC, num_subcores=NS),
            compiler_params=pltpu.CompilerParams(needs_layout_passes=False),
            scratch_types=[
                pltpu.VMEM((N_ATOMS,), jnp.int32),
                pltpu.VMEM((CHUNK,), jnp.int32),
                pltpu.VMEM((CHUNK,), jnp.int32),
                pltpu.VMEM((CHUNK,), jnp.int32),
                pltpu.VMEM((CHUNK,), jnp.float32),
                pltpu.VMEM((ZCHUNK,), jnp.float32),
                pltpu.VMEM_SHARED((HBINS,), jnp.float32),
                pltpu.SemaphoreType.DMA,
            ],
        )(_sc_hist_body))
    return _SC_HIST_CACHE[0](*args)


def _mm_body(h_ref, e_ref, o_ref):
    o_ref[...] = jnp.dot(h_ref[0] + h_ref[1], e_ref[...],
                         preferred_element_type=jnp.float32)


def _tc_matmul(h3, emb_pad):
    rows = 1000
    return pl.pallas_call(
        _mm_body,
        grid=(NUM_CYCLES // rows,),
        in_specs=[
            pl.BlockSpec((NC, rows, KPAD), lambda i: (0, i, 0)),
            pl.BlockSpec((KPAD, HIDDEN_DIM), lambda i: (0, 0)),
        ],
        out_specs=pl.BlockSpec((rows, HIDDEN_DIM), lambda i: (i, 0)),
        out_shape=jax.ShapeDtypeStruct((NUM_CYCLES, HIDDEN_DIM), jnp.float32),
    )(h3, emb_pad)


def kernel(x, atom_to_cycle, emb_weight):
    x = x.astype(jnp.int32)
    a2c_flat = atom_to_cycle.astype(jnp.int32).reshape(2 * N_MAP)
    zeros = jnp.zeros((ZCHUNK,), jnp.float32)
    ones = jnp.ones((CHUNK,), jnp.float32)
    hist = _sc_hist(x, a2c_flat, zeros, ones)
    h3 = hist.reshape(NC, NUM_CYCLES, KPAD)
    emb_pad = jnp.concatenate(
        [emb_weight.astype(jnp.float32),
         jnp.zeros((KPAD - VOCAB, HIDDEN_DIM), jnp.float32)], axis=0)
    return _tc_matmul(h3, emb_pad)


# speedup vs baseline: 43.7511x; 1.3068x over previous
import functools

import jax
import jax.numpy as jnp
from jax import lax
from jax.experimental import pallas as pl
from jax.experimental.pallas import tpu as pltpu
from jax.experimental.pallas import tpu_sc as plsc

N_ATOMS = 10000
N_MAP = 320000
NUM_CYCLES = 10000
HIDDEN_DIM = 128
VOCAB = 22
KPAD = 32
NC, NS, L = 2, 16, 16
NW = NC * NS
CHUNK = N_MAP // NW
HBINS = NUM_CYCLES * KPAD
ZCHUNK = HBINS // NS
NCHUNK = 5
SUB = CHUNK // NCHUNK
SUBV = SUB // L


def _sc_hist_body(x_hbm, a2c_hbm, z_hbm, ones_hbm, hist0_hbm, hist1_hbm,
                  x_v, src_v, dst_v, key_v, ones_v, stage_v, hist_sp,
                  sem_in):
    c = lax.axis_index("c")
    s = lax.axis_index("s")
    base = (c * NS + s) * CHUNK

    cp_z = pltpu.async_copy(z_hbm, stage_v, sem_in)
    cp_x = pltpu.async_copy(x_hbm, x_v, sem_in)
    cp_s = pltpu.async_copy(a2c_hbm.at[pl.ds(base, CHUNK)], src_v, sem_in)
    cp_d = pltpu.async_copy(a2c_hbm.at[pl.ds(N_MAP + base, CHUNK)], dst_v,
                            sem_in)
    cp_o = pltpu.async_copy(ones_hbm, ones_v, sem_in)
    cp_z.wait()
    pltpu.sync_copy(stage_v, hist_sp.at[pl.ds(s * ZCHUNK, ZCHUNK)])
    cp_x.wait()
    cp_s.wait()
    cp_d.wait()
    cp_o.wait()

    def body(i, carry):
        for u in range(5):
            off = i * 5 * L + u * L
            sv = src_v[pl.ds(off, L)]
            tv = plsc.load_gather(x_v, [sv])
            dv = dst_v[pl.ds(off, L)]
            key_v[pl.ds(off, L)] = dv * KPAD + tv
        return carry

    lax.fori_loop(0, CHUNK // (5 * L), body, 0)

    plsc.subcore_barrier()
    pltpu.sync_copy(ones_v, hist_sp.at[key_v], add=True)
    plsc.subcore_barrier()

    pltpu.sync_copy(hist_sp.at[pl.ds(s * ZCHUNK, ZCHUNK)], stage_v)

    @pl.when(c == 0)
    def _():
        pltpu.sync_copy(stage_v, hist0_hbm.at[pl.ds(s * ZCHUNK, ZCHUNK)])

    @pl.when(c == 1)
    def _():
        pltpu.sync_copy(stage_v, hist1_hbm.at[pl.ds(s * ZCHUNK, ZCHUNK)])


_SC_HIST_CACHE = []


def _sc_hist(*args):
    if not _SC_HIST_CACHE:
        _SC_HIST_CACHE.append(functools.partial(
            pl.kernel,
            out_type=(jax.ShapeDtypeStruct((HBINS,), jnp.float32),
                      jax.ShapeDtypeStruct((HBINS,), jnp.float32)),
            mesh=plsc.VectorSubcoreMesh(core_axis_name="c",
                                        subcore_axis_name="s",
                                        num_cores=NC, num_subcores=NS),
            compiler_params=pltpu.CompilerParams(needs_layout_passes=False),
            scratch_types=[
                pltpu.VMEM((N_ATOMS,), jnp.int32),
                pltpu.VMEM((CHUNK,), jnp.int32),
                pltpu.VMEM((CHUNK,), jnp.int32),
                pltpu.VMEM((CHUNK,), jnp.int32),
                pltpu.VMEM((CHUNK,), jnp.float32),
                pltpu.VMEM((ZCHUNK,), jnp.float32),
                pltpu.VMEM_SHARED((HBINS,), jnp.float32),
                pltpu.SemaphoreType.DMA,
            ],
        )(_sc_hist_body))
    return _SC_HIST_CACHE[0](*args)


GROUPS = HBINS // 128
GBLK = GROUPS
RBLK = GBLK * 4


def _mm_body(h0_ref, h1_ref, w_ref, o_ref):
    h = h0_ref[...] + h1_ref[...]
    o = jnp.dot(h, w_ref[...], preferred_element_type=jnp.float32)
    o_ref[...] = o.reshape(RBLK, HIDDEN_DIM)


def _tc_matmul(h0, h1, w):
    return pl.pallas_call(
        _mm_body,
        grid=(GROUPS // GBLK,),
        in_specs=[
            pl.BlockSpec((GBLK, 128), lambda i: (i, 0)),
            pl.BlockSpec((GBLK, 128), lambda i: (i, 0)),
            pl.BlockSpec((128, 4 * HIDDEN_DIM), lambda i: (0, 0)),
        ],
        out_specs=pl.BlockSpec((RBLK, HIDDEN_DIM), lambda i: (i, 0)),
        out_shape=jax.ShapeDtypeStruct((NUM_CYCLES, HIDDEN_DIM), jnp.float32),
    )(h0, h1, w)


def kernel(x, atom_to_cycle, emb_weight):
    x = x.astype(jnp.int32)
    a2c_flat = atom_to_cycle.astype(jnp.int32).reshape(2 * N_MAP)
    zeros = jnp.zeros((ZCHUNK,), jnp.float32)
    ones = jnp.ones((CHUNK,), jnp.float32)
    hist0, hist1 = _sc_hist(x, a2c_flat, zeros, ones)
    emb_pad = jnp.concatenate(
        [emb_weight.astype(jnp.float32),
         jnp.zeros((KPAD - VOCAB, HIDDEN_DIM), jnp.float32)], axis=0)
    w = jnp.kron(jnp.eye(4, dtype=jnp.float32), emb_pad)
    return _tc_matmul(hist0.reshape(GROUPS, 128), hist1.reshape(GROUPS, 128),
                      w)


# speedup vs baseline: 44.9297x; 1.0269x over previous
import functools

import jax
import jax.numpy as jnp
from jax import lax
from jax.experimental import pallas as pl
from jax.experimental.pallas import tpu as pltpu
from jax.experimental.pallas import tpu_sc as plsc

N_ATOMS = 10000
N_MAP = 320000
NUM_CYCLES = 10000
HIDDEN_DIM = 128
VOCAB = 22
KPAD = 32
NC, NS, L = 2, 16, 16
NW = NC * NS
CHUNK = N_MAP // NW
HBINS = NUM_CYCLES * KPAD
ZCHUNK = HBINS // NS
CHUNKA = 9984
TAIL = N_MAP - NW * CHUNKA
TAILB = NW * CHUNKA


def _sc_hist_body(x_hbm, a2c_hbm, z_hbm, ones_hbm, hist0_hbm, hist1_hbm,
                  x_v, av_v, key_v, ones_v, ae_v, keye_v, onese_v,
                  stage_v, hist_sp, sem_in):
    c = lax.axis_index("c")
    s = lax.axis_index("s")
    w = c * NS + s
    base = w * CHUNKA

    cp_z = pltpu.async_copy(z_hbm, stage_v, sem_in)
    cp_x = pltpu.async_copy(x_hbm, x_v, sem_in)
    cp_a = pltpu.async_copy(a2c_hbm.at[:, pl.ds(base, CHUNKA)], av_v, sem_in)
    cp_o = pltpu.async_copy(ones_hbm, ones_v, sem_in)

    @pl.when(w == 0)
    def _():
        pltpu.sync_copy(a2c_hbm.at[:, pl.ds(TAILB, TAIL)], ae_v)
        pltpu.sync_copy(ones_hbm.at[pl.ds(0, TAIL)], onese_v)

    cp_z.wait()
    pltpu.sync_copy(stage_v, hist_sp.at[pl.ds(s * ZCHUNK, ZCHUNK)])
    cp_x.wait()
    cp_a.wait()
    cp_o.wait()

    def body(i, carry):
        for u in range(4):
            off = i * 4 * L + u * L
            sv = av_v[0, pl.ds(off, L)]
            tv = plsc.load_gather(x_v, [sv])
            dv = av_v[1, pl.ds(off, L)]
            key_v[pl.ds(off, L)] = dv * KPAD + tv
        return carry

    lax.fori_loop(0, CHUNKA // (4 * L), body, 0)

    @pl.when(w == 0)
    def _():
        def tbody(i, carry):
            for u in range(4):
                off = i * 4 * L + u * L
                sv = ae_v[0, pl.ds(off, L)]
                tv = plsc.load_gather(x_v, [sv])
                dv = ae_v[1, pl.ds(off, L)]
                keye_v[pl.ds(off, L)] = dv * KPAD + tv
            return carry

        lax.fori_loop(0, TAIL // (4 * L), tbody, 0)

    plsc.subcore_barrier()
    pltpu.sync_copy(ones_v, hist_sp.at[key_v], add=True)

    @pl.when(w == 0)
    def _():
        pltpu.sync_copy(onese_v, hist_sp.at[keye_v], add=True)

    plsc.subcore_barrier()

    pltpu.sync_copy(hist_sp.at[pl.ds(s * ZCHUNK, ZCHUNK)], stage_v)

    @pl.when(c == 0)
    def _():
        pltpu.sync_copy(stage_v, hist0_hbm.at[pl.ds(s * ZCHUNK, ZCHUNK)])

    @pl.when(c == 1)
    def _():
        pltpu.sync_copy(stage_v, hist1_hbm.at[pl.ds(s * ZCHUNK, ZCHUNK)])


_SC_HIST_CACHE = []


def _sc_hist(*args):
    if not _SC_HIST_CACHE:
        _SC_HIST_CACHE.append(functools.partial(
            pl.kernel,
            out_type=(jax.ShapeDtypeStruct((HBINS,), jnp.float32),
                      jax.ShapeDtypeStruct((HBINS,), jnp.float32)),
            mesh=plsc.VectorSubcoreMesh(core_axis_name="c",
                                        subcore_axis_name="s",
                                        num_cores=NC, num_subcores=NS),
            compiler_params=pltpu.CompilerParams(needs_layout_passes=False),
            scratch_types=[
                pltpu.VMEM((N_ATOMS,), jnp.int32),
                pltpu.VMEM((2, CHUNKA), jnp.int32),
                pltpu.VMEM((CHUNKA,), jnp.int32),
                pltpu.VMEM((CHUNKA,), jnp.float32),
                pltpu.VMEM((2, TAIL), jnp.int32),
                pltpu.VMEM((TAIL,), jnp.int32),
                pltpu.VMEM((TAIL,), jnp.float32),
                pltpu.VMEM((ZCHUNK,), jnp.float32),
                pltpu.VMEM_SHARED((HBINS,), jnp.float32),
                pltpu.SemaphoreType.DMA,
            ],
        )(_sc_hist_body))
    return _SC_HIST_CACHE[0](*args)


GROUPS = HBINS // 128
GBLK = GROUPS
RBLK = GBLK * 4


def _mm_body(h0_ref, h1_ref, w_ref, o_ref):
    h = h0_ref[...] + h1_ref[...]
    o = jnp.dot(h, w_ref[...], preferred_element_type=jnp.float32)
    o_ref[...] = o.reshape(RBLK, HIDDEN_DIM)


def _tc_matmul(h0, h1, w):
    return pl.pallas_call(
        _mm_body,
        grid=(GROUPS // GBLK,),
        in_specs=[
            pl.BlockSpec((GBLK, 128), lambda i: (i, 0)),
            pl.BlockSpec((GBLK, 128), lambda i: (i, 0)),
            pl.BlockSpec((128, 4 * HIDDEN_DIM), lambda i: (0, 0)),
        ],
        out_specs=pl.BlockSpec((RBLK, HIDDEN_DIM), lambda i: (i, 0)),
        out_shape=jax.ShapeDtypeStruct((NUM_CYCLES, HIDDEN_DIM), jnp.float32),
    )(h0, h1, w)


def kernel(x, atom_to_cycle, emb_weight):
    x = x.astype(jnp.int32)
    a2c = atom_to_cycle.astype(jnp.int32)
    zeros = jnp.zeros((ZCHUNK,), jnp.float32)
    ones = jnp.ones((CHUNKA,), jnp.float32)
    hist0, hist1 = _sc_hist(x, a2c, zeros, ones)
    emb_pad = jnp.concatenate(
        [emb_weight.astype(jnp.float32),
         jnp.zeros((KPAD - VOCAB, HIDDEN_DIM), jnp.float32)], axis=0)
    w = jnp.kron(jnp.eye(4, dtype=jnp.float32), emb_pad)
    return _tc_matmul(hist0.reshape(GROUPS, 128), hist1.reshape(GROUPS, 128),
                      w)


# speedup vs baseline: 47.1219x; 1.0488x over previous
import functools

import jax
import jax.numpy as jnp
from jax import lax
from jax.experimental import pallas as pl
from jax.experimental.pallas import tpu as pltpu
from jax.experimental.pallas import tpu_sc as plsc

N_ATOMS = 10000
N_MAP = 320000
NUM_CYCLES = 10000
HIDDEN_DIM = 128
VOCAB = 22
KPAD = 32
NC, NS, L = 2, 16, 16
NW = NC * NS
CHUNK = N_MAP // NW
HBINS = NUM_CYCLES * KPAD
ZCHUNK = HBINS // NS
CHUNKA = 9984
TAIL = N_MAP - NW * CHUNKA
TAILB = NW * CHUNKA


def _sc_hist_body(x_hbm, a2c_hbm, z_hbm, ones_hbm, hist0_hbm, hist1_hbm,
                  x_v, av_v, key_v, ones_v, ae_v, keye_v, onese_v,
                  stage_v, hist_sp, sem_in, sem_sc):
    c = lax.axis_index("c")
    s = lax.axis_index("s")
    w = c * NS + s
    base = w * CHUNKA

    cp_z = pltpu.async_copy(z_hbm, stage_v, sem_in)
    cp_x = pltpu.async_copy(x_hbm, x_v, sem_in)
    cp_a = pltpu.async_copy(a2c_hbm.at[:, pl.ds(base, CHUNKA)], av_v, sem_in)
    cp_o = pltpu.async_copy(ones_hbm, ones_v, sem_in)

    @pl.when(w == 0)
    def _():
        pltpu.sync_copy(a2c_hbm.at[:, pl.ds(TAILB, TAIL)], ae_v)
        pltpu.sync_copy(ones_hbm.at[pl.ds(0, TAIL)], onese_v)

    cp_z.wait()
    pltpu.sync_copy(stage_v, hist_sp.at[pl.ds(s * ZCHUNK, ZCHUNK)])
    cp_x.wait()
    cp_a.wait()
    cp_o.wait()

    half = CHUNKA // 2

    def make_body(hbase):
        def body(i, carry):
            for u in range(4):
                off = hbase + i * 4 * L + u * L
                sv = av_v[0, pl.ds(off, L)]
                tv = plsc.load_gather(x_v, [sv])
                dv = av_v[1, pl.ds(off, L)]
                key_v[pl.ds(off, L)] = dv * KPAD + tv
            return carry
        return body

    lax.fori_loop(0, half // (4 * L), make_body(0), 0)
    plsc.subcore_barrier()
    fire0 = pltpu.async_copy(ones_v.at[pl.ds(0, half)],
                             hist_sp.at[key_v.at[pl.ds(0, half)]],
                             sem_sc, add=True)
    lax.fori_loop(0, half // (4 * L), make_body(half), 0)
    fire1 = pltpu.async_copy(ones_v.at[pl.ds(half, half)],
                             hist_sp.at[key_v.at[pl.ds(half, half)]],
                             sem_sc, add=True)

    @pl.when(w == 0)
    def _():
        def tbody(i, carry):
            for u in range(4):
                off = i * 4 * L + u * L
                sv = ae_v[0, pl.ds(off, L)]
                tv = plsc.load_gather(x_v, [sv])
                dv = ae_v[1, pl.ds(off, L)]
                keye_v[pl.ds(off, L)] = dv * KPAD + tv
            return carry

        lax.fori_loop(0, TAIL // (4 * L), tbody, 0)
        pltpu.sync_copy(onese_v, hist_sp.at[keye_v], add=True)

    fire0.wait()
    fire1.wait()
    plsc.subcore_barrier()

    pltpu.sync_copy(hist_sp.at[pl.ds(s * ZCHUNK, ZCHUNK)], stage_v)

    @pl.when(c == 0)
    def _():
        pltpu.sync_copy(stage_v, hist0_hbm.at[pl.ds(s * ZCHUNK, ZCHUNK)])

    @pl.when(c == 1)
    def _():
        pltpu.sync_copy(stage_v, hist1_hbm.at[pl.ds(s * ZCHUNK, ZCHUNK)])


_SC_HIST_CACHE = []


def _sc_hist(*args):
    if not _SC_HIST_CACHE:
        _SC_HIST_CACHE.append(functools.partial(
            pl.kernel,
            out_type=(jax.ShapeDtypeStruct((HBINS,), jnp.float32),
                      jax.ShapeDtypeStruct((HBINS,), jnp.float32)),
            mesh=plsc.VectorSubcoreMesh(core_axis_name="c",
                                        subcore_axis_name="s",
                                        num_cores=NC, num_subcores=NS),
            compiler_params=pltpu.CompilerParams(needs_layout_passes=False),
            scratch_types=[
                pltpu.VMEM((N_ATOMS,), jnp.int32),
                pltpu.VMEM((2, CHUNKA), jnp.int32),
                pltpu.VMEM((CHUNKA,), jnp.int32),
                pltpu.VMEM((CHUNKA,), jnp.float32),
                pltpu.VMEM((2, TAIL), jnp.int32),
                pltpu.VMEM((TAIL,), jnp.int32),
                pltpu.VMEM((TAIL,), jnp.float32),
                pltpu.VMEM((ZCHUNK,), jnp.float32),
                pltpu.VMEM_SHARED((HBINS,), jnp.float32),
                pltpu.SemaphoreType.DMA,
                pltpu.SemaphoreType.DMA,
            ],
        )(_sc_hist_body))
    return _SC_HIST_CACHE[0](*args)


GROUPS = HBINS // 128
GBLK = GROUPS
RBLK = GBLK * 4


def _mm_body(h0_ref, h1_ref, w_ref, o_ref):
    h = h0_ref[...] + h1_ref[...]
    o = jnp.dot(h, w_ref[...], preferred_element_type=jnp.float32)
    o_ref[...] = o.reshape(RBLK, HIDDEN_DIM)


def _tc_matmul(h0, h1, w):
    return pl.pallas_call(
        _mm_body,
        grid=(GROUPS // GBLK,),
        in_specs=[
            pl.BlockSpec((GBLK, 128), lambda i: (i, 0)),
            pl.BlockSpec((GBLK, 128), lambda i: (i, 0)),
            pl.BlockSpec((128, 4 * HIDDEN_DIM), lambda i: (0, 0)),
        ],
        out_specs=pl.BlockSpec((RBLK, HIDDEN_DIM), lambda i: (i, 0)),
        out_shape=jax.ShapeDtypeStruct((NUM_CYCLES, HIDDEN_DIM), jnp.float32),
    )(h0, h1, w)


def kernel(x, atom_to_cycle, emb_weight):
    x = x.astype(jnp.int32)
    a2c = atom_to_cycle.astype(jnp.int32)
    zeros = jnp.zeros((ZCHUNK,), jnp.float32)
    ones = jnp.ones((CHUNKA,), jnp.float32)
    hist0, hist1 = _sc_hist(x, a2c, zeros, ones)
    emb_pad = jnp.concatenate(
        [emb_weight.astype(jnp.float32),
         jnp.zeros((KPAD - VOCAB, HIDDEN_DIM), jnp.float32)], axis=0)
    w = jnp.kron(jnp.eye(4, dtype=jnp.float32), emb_pad)
    return _tc_matmul(hist0.reshape(GROUPS, 128), hist1.reshape(GROUPS, 128),
                      w)


# speedup vs baseline: 48.4213x; 1.0276x over previous
import functools

import jax
import jax.numpy as jnp
from jax import lax
from jax.experimental import pallas as pl
from jax.experimental.pallas import tpu as pltpu
from jax.experimental.pallas import tpu_sc as plsc

N_ATOMS = 10000
N_MAP = 320000
NUM_CYCLES = 10000
HIDDEN_DIM = 128
VOCAB = 22
KPAD = 32
NC, NS, L = 2, 16, 16
NW = NC * NS
CHUNK = N_MAP // NW
HBINS = NUM_CYCLES * KPAD
ZCHUNK = HBINS // NS
CHUNKA = 9984
TAIL = N_MAP - NW * CHUNKA
TAILB = NW * CHUNKA


def _sc_hist_body(x_hbm, a2c_hbm, z_hbm, hist0_hbm, hist1_hbm,
                  x_v, av_v, key_v, ones_v, ae_v, keye_v, onese_v,
                  stage_v, hist_sp, sem_in, sem_sc):
    c = lax.axis_index("c")
    s = lax.axis_index("s")
    w = c * NS + s
    base = w * CHUNKA

    cp_z = pltpu.async_copy(z_hbm.at[pl.ds(0, ZCHUNK)], stage_v, sem_in)
    cp_x = pltpu.async_copy(x_hbm, x_v, sem_in)
    cp_a = pltpu.async_copy(a2c_hbm.at[:, pl.ds(base, CHUNKA)], av_v, sem_in)
    cp_o = pltpu.async_copy(z_hbm.at[pl.ds(ZCHUNK, CHUNKA)], ones_v, sem_in)

    @pl.when(w == 0)
    def _():
        pltpu.sync_copy(a2c_hbm.at[:, pl.ds(TAILB, TAIL)], ae_v)
        pltpu.sync_copy(z_hbm.at[pl.ds(ZCHUNK, TAIL)], onese_v)

    cp_z.wait()
    pltpu.sync_copy(stage_v, hist_sp.at[pl.ds(s * ZCHUNK, ZCHUNK)])
    cp_x.wait()
    cp_a.wait()
    cp_o.wait()

    quart = CHUNKA // 4

    def make_body(hbase):
        def body(i, carry):
            for u in range(4):
                off = hbase + i * 4 * L + u * L
                sv = av_v[0, pl.ds(off, L)]
                tv = plsc.load_gather(x_v, [sv])
                dv = av_v[1, pl.ds(off, L)]
                key_v[pl.ds(off, L)] = dv * KPAD + tv
            return carry
        return body

    fires = []
    lax.fori_loop(0, quart // (4 * L), make_body(0), 0)
    plsc.subcore_barrier()
    fires.append(pltpu.async_copy(ones_v.at[pl.ds(0, quart)],
                                  hist_sp.at[key_v.at[pl.ds(0, quart)]],
                                  sem_sc, add=True))
    for q in range(1, 4):
        lax.fori_loop(0, quart // (4 * L), make_body(q * quart), 0)
        fires.append(pltpu.async_copy(
            ones_v.at[pl.ds(q * quart, quart)],
            hist_sp.at[key_v.at[pl.ds(q * quart, quart)]],
            sem_sc, add=True))

    @pl.when(w == 0)
    def _():
        def tbody(i, carry):
            for u in range(4):
                off = i * 4 * L + u * L
                sv = ae_v[0, pl.ds(off, L)]
                tv = plsc.load_gather(x_v, [sv])
                dv = ae_v[1, pl.ds(off, L)]
                keye_v[pl.ds(off, L)] = dv * KPAD + tv
            return carry

        lax.fori_loop(0, TAIL // (4 * L), tbody, 0)
        pltpu.sync_copy(onese_v, hist_sp.at[keye_v], add=True)

    for f in fires:
        f.wait()
    plsc.subcore_barrier()

    zh = ZCHUNK // 2
    hist_hbm = (hist0_hbm, hist1_hbm)
    for ci in range(NC):
        @pl.when(c == ci)
        def _(ci=ci):
            pltpu.sync_copy(hist_sp.at[pl.ds(s * ZCHUNK, zh)],
                            stage_v.at[pl.ds(0, zh)])
            cp = pltpu.async_copy(stage_v.at[pl.ds(0, zh)],
                                  hist_hbm[ci].at[pl.ds(s * ZCHUNK, zh)],
                                  sem_sc)
            pltpu.sync_copy(hist_sp.at[pl.ds(s * ZCHUNK + zh, zh)],
                            stage_v.at[pl.ds(zh, zh)])
            cp.wait()
            pltpu.sync_copy(stage_v.at[pl.ds(zh, zh)],
                            hist_hbm[ci].at[pl.ds(s * ZCHUNK + zh, zh)])


_SC_HIST_CACHE = []


def _sc_hist(*args):
    if not _SC_HIST_CACHE:
        _SC_HIST_CACHE.append(functools.partial(
            pl.kernel,
            out_type=(jax.ShapeDtypeStruct((HBINS,), jnp.float32),
                      jax.ShapeDtypeStruct((HBINS,), jnp.float32)),
            mesh=plsc.VectorSubcoreMesh(core_axis_name="c",
                                        subcore_axis_name="s",
                                        num_cores=NC, num_subcores=NS),
            compiler_params=pltpu.CompilerParams(needs_layout_passes=False),
            scratch_types=[
                pltpu.VMEM((N_ATOMS,), jnp.int32),
                pltpu.VMEM((2, CHUNKA), jnp.int32),
                pltpu.VMEM((CHUNKA,), jnp.int32),
                pltpu.VMEM((CHUNKA,), jnp.float32),
                pltpu.VMEM((2, TAIL), jnp.int32),
                pltpu.VMEM((TAIL,), jnp.int32),
                pltpu.VMEM((TAIL,), jnp.float32),
                pltpu.VMEM((ZCHUNK,), jnp.float32),
                pltpu.VMEM_SHARED((HBINS,), jnp.float32),
                pltpu.SemaphoreType.DMA,
                pltpu.SemaphoreType.DMA,
            ],
        )(_sc_hist_body))
    return _SC_HIST_CACHE[0](*args)


GROUPS = HBINS // 128
GBLK = GROUPS
RBLK = GBLK * 4


def _mm_body(h0_ref, h1_ref, w_ref, o_ref):
    h = h0_ref[...] + h1_ref[...]
    o = jnp.dot(h, w_ref[...], preferred_element_type=jnp.float32)
    o_ref[...] = o.reshape(RBLK, HIDDEN_DIM)


def _tc_matmul(h0, h1, w):
    return pl.pallas_call(
        _mm_body,
        grid=(GROUPS // GBLK,),
        in_specs=[
            pl.BlockSpec((GBLK, 128), lambda i: (i, 0)),
            pl.BlockSpec((GBLK, 128), lambda i: (i, 0)),
            pl.BlockSpec((128, 4 * HIDDEN_DIM), lambda i: (0, 0)),
        ],
        out_specs=pl.BlockSpec((RBLK, HIDDEN_DIM), lambda i: (i, 0)),
        out_shape=jax.ShapeDtypeStruct((NUM_CYCLES, HIDDEN_DIM), jnp.float32),
    )(h0, h1, w)


def kernel(x, atom_to_cycle, emb_weight):
    x = x.astype(jnp.int32)
    a2c = atom_to_cycle.astype(jnp.int32)
    zeros_ones = jnp.concatenate([jnp.zeros((ZCHUNK,), jnp.float32),
                                  jnp.ones((CHUNKA,), jnp.float32)])
    hist0, hist1 = _sc_hist(x, a2c, zeros_ones)
    emb_pad = jnp.concatenate(
        [emb_weight.astype(jnp.float32),
         jnp.zeros((KPAD - VOCAB, HIDDEN_DIM), jnp.float32)], axis=0)
    w = jnp.kron(jnp.eye(4, dtype=jnp.float32), emb_pad)
    return _tc_matmul(hist0.reshape(GROUPS, 128), hist1.reshape(GROUPS, 128),
                      w)


# speedup vs baseline: 48.8074x; 1.0080x over previous
import functools

import jax
import jax.numpy as jnp
from jax import lax
from jax.experimental import pallas as pl
from jax.experimental.pallas import tpu as pltpu
from jax.experimental.pallas import tpu_sc as plsc

N_ATOMS = 10000
N_MAP = 320000
NUM_CYCLES = 10000
HIDDEN_DIM = 128
VOCAB = 22
KPAD = 32
NC, NS, L = 2, 16, 16
NW = NC * NS
CHUNK = N_MAP // NW
HBINS = NUM_CYCLES * KPAD
ZCHUNK = HBINS // NS
CHUNKA = 9984
TAIL = N_MAP - NW * CHUNKA
TAILB = NW * CHUNKA


def _sc_hist_body(x_hbm, a2c_hbm, z_hbm, hist0_hbm, hist1_hbm,
                  x_v, av_v, key_v, ones_v, ae_v, keye_v, onese_v,
                  stage_v, hist_sp, sem_x, sem_z, sem_zs, sem_a0, sem_a1,
                  sem_o, sem_sc):
    c = lax.axis_index("c")
    s = lax.axis_index("s")
    w = c * NS + s
    base = w * CHUNKA

    halfa = CHUNKA // 2
    cp_x = pltpu.async_copy(x_hbm, x_v, sem_x)
    cp_a0 = pltpu.async_copy(a2c_hbm.at[:, pl.ds(base, halfa)],
                             av_v.at[:, pl.ds(0, halfa)], sem_a0)
    cp_z = pltpu.async_copy(z_hbm.at[pl.ds(0, ZCHUNK)], stage_v, sem_z)
    cp_a1 = pltpu.async_copy(a2c_hbm.at[:, pl.ds(base + halfa, halfa)],
                             av_v.at[:, pl.ds(halfa, halfa)], sem_a1)
    cp_o = pltpu.async_copy(z_hbm.at[pl.ds(ZCHUNK, CHUNKA)], ones_v, sem_o)

    @pl.when(w == 0)
    def _():
        pltpu.sync_copy(a2c_hbm.at[:, pl.ds(TAILB, TAIL)], ae_v)
        pltpu.sync_copy(z_hbm.at[pl.ds(ZCHUNK, TAIL)], onese_v)

    cp_z.wait()
    cp_zs = pltpu.async_copy(stage_v, hist_sp.at[pl.ds(s * ZCHUNK, ZCHUNK)],
                             sem_zs)
    cp_x.wait()
    cp_a0.wait()

    quart = CHUNKA // 4

    def make_body(hbase):
        def body(i, carry):
            for u in range(4):
                off = hbase + i * 4 * L + u * L
                sv = av_v[0, pl.ds(off, L)]
                tv = plsc.load_gather(x_v, [sv])
                dv = av_v[1, pl.ds(off, L)]
                key_v[pl.ds(off, L)] = dv * KPAD + tv
            return carry
        return body

    fires = []
    lax.fori_loop(0, quart // (4 * L), make_body(0), 0)
    cp_o.wait()
    cp_zs.wait()
    plsc.subcore_barrier()
    fires.append(pltpu.async_copy(ones_v.at[pl.ds(0, quart)],
                                  hist_sp.at[key_v.at[pl.ds(0, quart)]],
                                  sem_sc, add=True))
    lax.fori_loop(0, quart // (4 * L), make_body(quart), 0)
    fires.append(pltpu.async_copy(ones_v.at[pl.ds(quart, quart)],
                                  hist_sp.at[key_v.at[pl.ds(quart, quart)]],
                                  sem_sc, add=True))
    cp_a1.wait()
    for q in range(2, 4):
        lax.fori_loop(0, quart // (4 * L), make_body(q * quart), 0)
        fires.append(pltpu.async_copy(
            ones_v.at[pl.ds(q * quart, quart)],
            hist_sp.at[key_v.at[pl.ds(q * quart, quart)]],
            sem_sc, add=True))

    @pl.when(w == 0)
    def _():
        def tbody(i, carry):
            for u in range(4):
                off = i * 4 * L + u * L
                sv = ae_v[0, pl.ds(off, L)]
                tv = plsc.load_gather(x_v, [sv])
                dv = ae_v[1, pl.ds(off, L)]
                keye_v[pl.ds(off, L)] = dv * KPAD + tv
            return carry

        lax.fori_loop(0, TAIL // (4 * L), tbody, 0)
        pltpu.sync_copy(onese_v, hist_sp.at[keye_v], add=True)

    for f in fires:
        f.wait()
    plsc.subcore_barrier()

    zh = ZCHUNK // 2
    hist_hbm = (hist0_hbm, hist1_hbm)
    for ci in range(NC):
        @pl.when(c == ci)
        def _(ci=ci):
            pltpu.sync_copy(hist_sp.at[pl.ds(s * ZCHUNK, zh)],
                            stage_v.at[pl.ds(0, zh)])
            cp = pltpu.async_copy(stage_v.at[pl.ds(0, zh)],
                                  hist_hbm[ci].at[pl.ds(s * ZCHUNK, zh)],
                                  sem_sc)
            pltpu.sync_copy(hist_sp.at[pl.ds(s * ZCHUNK + zh, zh)],
                            stage_v.at[pl.ds(zh, zh)])
            cp.wait()
            pltpu.sync_copy(stage_v.at[pl.ds(zh, zh)],
                            hist_hbm[ci].at[pl.ds(s * ZCHUNK + zh, zh)])


_SC_HIST_CACHE = []


def _sc_hist(*args):
    if not _SC_HIST_CACHE:
        _SC_HIST_CACHE.append(functools.partial(
            pl.kernel,
            out_type=(jax.ShapeDtypeStruct((HBINS,), jnp.float32),
                      jax.ShapeDtypeStruct((HBINS,), jnp.float32)),
            mesh=plsc.VectorSubcoreMesh(core_axis_name="c",
                                        subcore_axis_name="s",
                                        num_cores=NC, num_subcores=NS),
            compiler_params=pltpu.CompilerParams(needs_layout_passes=False),
            scratch_types=[
                pltpu.VMEM((N_ATOMS,), jnp.int32),
                pltpu.VMEM((2, CHUNKA), jnp.int32),
                pltpu.VMEM((CHUNKA,), jnp.int32),
                pltpu.VMEM((CHUNKA,), jnp.float32),
                pltpu.VMEM((2, TAIL), jnp.int32),
                pltpu.VMEM((TAIL,), jnp.int32),
                pltpu.VMEM((TAIL,), jnp.float32),
                pltpu.VMEM((ZCHUNK,), jnp.float32),
                pltpu.VMEM_SHARED((HBINS,), jnp.float32),
                pltpu.SemaphoreType.DMA,
                pltpu.SemaphoreType.DMA,
                pltpu.SemaphoreType.DMA,
                pltpu.SemaphoreType.DMA,
                pltpu.SemaphoreType.DMA,
                pltpu.SemaphoreType.DMA,
                pltpu.SemaphoreType.DMA,
            ],
        )(_sc_hist_body))
    return _SC_HIST_CACHE[0](*args)


GROUPS = HBINS // 128
GBLK = GROUPS
RBLK = GBLK * 4


def _mm_body(h0_ref, h1_ref, w_ref, o_ref):
    h = h0_ref[...] + h1_ref[...]
    o = jnp.dot(h, w_ref[...], preferred_element_type=jnp.float32)
    o_ref[...] = o.reshape(RBLK, HIDDEN_DIM)


def _tc_matmul(h0, h1, w):
    return pl.pallas_call(
        _mm_body,
        grid=(GROUPS // GBLK,),
        in_specs=[
            pl.BlockSpec((GBLK, 128), lambda i: (i, 0)),
            pl.BlockSpec((GBLK, 128), lambda i: (i, 0)),
            pl.BlockSpec((128, 4 * HIDDEN_DIM), lambda i: (0, 0)),
        ],
        out_specs=pl.BlockSpec((RBLK, HIDDEN_DIM), lambda i: (i, 0)),
        out_shape=jax.ShapeDtypeStruct((NUM_CYCLES, HIDDEN_DIM), jnp.float32),
    )(h0, h1, w)


def kernel(x, atom_to_cycle, emb_weight):
    x = x.astype(jnp.int32)
    a2c = atom_to_cycle.astype(jnp.int32)
    zeros_ones = jnp.concatenate([jnp.zeros((ZCHUNK,), jnp.float32),
                                  jnp.ones((CHUNKA,), jnp.float32)])
    hist0, hist1 = _sc_hist(x, a2c, zeros_ones)
    emb_pad = jnp.concatenate(
        [emb_weight.astype(jnp.float32),
         jnp.zeros((KPAD - VOCAB, HIDDEN_DIM), jnp.float32)], axis=0)
    w = jnp.kron(jnp.eye(4, dtype=jnp.float32), emb_pad)
    return _tc_matmul(hist0.reshape(GROUPS, 128), hist1.reshape(GROUPS, 128),
                      w)
